# baseline (speedup 1.0000x reference)
import functools

import jax
import jax.numpy as jnp
from jax import lax
from jax.experimental import pallas as pl
from jax.experimental.pallas import tpu as pltpu
from jax.experimental.pallas import tpu_sc as plsc

_EPS = 1e-5
_HALF = 128
_B = 125
_TILES = 16


def _mm_stack_body(x_ref, w_ref, out_ref):
    h = lax.dot_general(x_ref[...], w_ref[...], (((1,), (1,)), ((), ())),
                        preferred_element_type=jnp.float32)
    out_ref[0] = h[:, :_HALF]
    out_ref[1] = h[:, _HALF:]


def _mm_stack(x, w, bn):
    n, d = x.shape
    return pl.pallas_call(
        _mm_stack_body,
        grid=(n // bn,),
        in_specs=[
            pl.BlockSpec((bn, d), lambda i: (i, 0)),
            pl.BlockSpec(w.shape, lambda i: (0, 0)),
        ],
        out_specs=pl.BlockSpec((2, bn, _HALF), lambda i: (0, i, 0)),
        out_shape=jax.ShapeDtypeStruct((2, n, _HALF), jnp.float32),
    )(x, w)


def _ln_relu(a_ref, g_ref, b_ref):
    a = jnp.concatenate([a_ref[0], a_ref[1]], axis=-1)
    mu = jnp.mean(a, axis=-1, keepdims=True)
    var = jnp.mean(jnp.square(a - mu), axis=-1, keepdims=True)
    hn = (a - mu) * lax.rsqrt(var + _EPS) * g_ref[...] + b_ref[...]
    return jnp.maximum(hn, 0.0)


def _ln_mm_stack_body(a_ref, g_ref, b_ref, w_ref, out_ref):
    h = _ln_relu(a_ref, g_ref, b_ref)
    o = lax.dot_general(h, w_ref[...], (((1,), (1,)), ((), ())),
                        preferred_element_type=jnp.float32)
    out_ref[0] = o[:, :_HALF]
    out_ref[1] = o[:, _HALF:]


def _ln_mm_stack(a, g, b, w, bn, n):
    return pl.pallas_call(
        _ln_mm_stack_body,
        grid=(n // bn,),
        in_specs=[
            pl.BlockSpec((2, bn, _HALF), lambda i: (0, i, 0)),
            pl.BlockSpec(g.shape, lambda i: (0, 0)),
            pl.BlockSpec(b.shape, lambda i: (0, 0)),
            pl.BlockSpec(w.shape, lambda i: (0, 0)),
        ],
        out_specs=pl.BlockSpec((2, bn, _HALF), lambda i: (0, i, 0)),
        out_shape=jax.ShapeDtypeStruct((2, n, _HALF), jnp.float32),
    )(a, g, b, w)


def _final_body(a_ref, g_ref, b_ref, w_ref, bo_ref, out_ref):
    h = _ln_relu(a_ref, g_ref, b_ref)
    logits = lax.dot_general(h, w_ref[...], (((1,), (1,)), ((), ())),
                             preferred_element_type=jnp.float32) + bo_ref[...]
    m = jnp.max(logits, axis=-1, keepdims=True)
    lse = jnp.log(jnp.sum(jnp.exp(logits - m), axis=-1, keepdims=True)) + m
    out_ref[...] = logits - lse


def _final(a, g, b, w, bo, bn, n):
    c = w.shape[0]
    return pl.pallas_call(
        _final_body,
        grid=(n // bn,),
        in_specs=[
            pl.BlockSpec((2, bn, _HALF), lambda i: (0, i, 0)),
            pl.BlockSpec(g.shape, lambda i: (0, 0)),
            pl.BlockSpec(b.shape, lambda i: (0, 0)),
            pl.BlockSpec(w.shape, lambda i: (0, 0)),
            pl.BlockSpec(bo.shape, lambda i: (0, 0)),
        ],
        out_specs=pl.BlockSpec((bn, c), lambda i: (i, 0)),
        out_shape=jax.ShapeDtypeStruct((n, c), jnp.float32),
    )(a, g, b, w, bo)


_NBUF = 2


def _aggregate(h_stacked, srcoff, dst3, n, n_pad):
    ngrp = dst3.shape[0] // _TILES
    rows_per_tile = n_pad // _TILES
    zrows = 32
    nz = rows_per_tile // zrows
    assert rows_per_tile % zrows == 0 and _NBUF == 2 and ngrp >= 3

    mesh = plsc.VectorSubcoreMesh(core_axis_name="c", subcore_axis_name="s")

    @functools.partial(
        pl.kernel,
        mesh=mesh,
        out_type=jax.ShapeDtypeStruct((2, n_pad, _HALF), jnp.float32),
        scratch_types=[
            pltpu.VMEM((3, _NBUF, _B), jnp.int32),
            pltpu.VMEM((3, _NBUF, _B), jnp.int32),
            pltpu.VMEM((_NBUF, _B, _HALF), jnp.float32),
            pltpu.VMEM((zrows, _HALF), jnp.float32),
            pltpu.VMEM_SHARED((n_pad, _HALF), jnp.float32),
        ] + [pltpu.SemaphoreType.DMA((3,))] + [pltpu.SemaphoreType.DMA] * 4,
    )
    def agg(h_hbm, src_hbm, dst_hbm, out_hbm, src_g, dst_g, rows_v, zero_v,
            acc_sh, sem_i, sg0, sg1, ss0, ss1):
        c = lax.axis_index("c")
        s = lax.axis_index("s")
        srow0 = (c * _TILES + s) * ngrp
        drow0 = s * ngrp
        sem_g = [sg0, sg1]
        sem_s = [ss0, ss1]

        def idx_issue(grp, slot):
            pltpu.async_copy(src_hbm.at[srow0 + grp], src_g.at[slot],
                             sem_i.at[slot])
            pltpu.async_copy(dst_hbm.at[drow0 + grp], dst_g.at[slot],
                             sem_i.at[slot])

        def idx_wait(grp, slot):
            pltpu.make_async_copy(src_hbm.at[srow0 + grp], src_g.at[slot],
                                  sem_i.at[slot]).wait()
            pltpu.make_async_copy(dst_hbm.at[drow0 + grp], dst_g.at[slot],
                                  sem_i.at[slot]).wait()

        def gather_issue(slot, b):
            pltpu.async_copy(h_hbm.at[src_g.at[slot, b]], rows_v.at[b],
                             sem_g[b])

        def gather_wait(slot, b):
            pltpu.make_async_copy(h_hbm.at[src_g.at[slot, b]], rows_v.at[b],
                                  sem_g[b]).wait()

        def scatter_issue(slot, b):
            pltpu.async_copy(rows_v.at[b], acc_sh.at[dst_g.at[slot, b]],
                             sem_s[b], add=True)

        def scatter_wait(slot, b):
            pltpu.make_async_copy(rows_v.at[b], acc_sh.at[dst_g.at[slot, b]],
                                  sem_s[b]).wait()

        for grp in range(3):
            idx_issue(grp, grp)

        def zrow(i, _):
            def zcol(j, _):
                zero_v[i, pl.ds(j * 16, 16)] = jnp.zeros((16,), jnp.float32)
                return 0
            return lax.fori_loop(0, _HALF // 16, zcol, 0)
        lax.fori_loop(0, zrows, zrow, 0)
        for k in range(nz):
            pltpu.sync_copy(
                zero_v,
                acc_sh.at[pl.ds(s * rows_per_tile + k * zrows, zrows)])

        idx_wait(0, 0)
        gather_issue(0, 0)
        plsc.subcore_barrier()

        def group(g, _):
            r = lax.rem(g, 3)
            r1 = lax.rem(g + 1, 3)
            r2 = lax.rem(g + 2, 3)

            gather_wait(r, 0)
            scatter_issue(r, 0)

            @pl.when(g > 0)
            def _():
                scatter_wait(lax.rem(g + 2, 3), 1)

                @pl.when(g + 2 < ngrp)
                def _():
                    idx_issue(g + 2, r2)
            gather_issue(r, 1)

            gather_wait(r, 1)
            scatter_issue(r, 1)
            scatter_wait(r, 0)

            @pl.when(g + 1 < ngrp)
            def _():
                idx_wait(g + 1, r1)
                gather_issue(r1, 0)
            return 0
        lax.fori_loop(0, ngrp, group, 0)
        scatter_wait(lax.rem(ngrp - 1, 3), 1)
        plsc.subcore_barrier()

        pltpu.sync_copy(
            acc_sh.at[pl.ds(s * rows_per_tile, rows_per_tile)],
            out_hbm.at[c, pl.ds(s * rows_per_tile, rows_per_tile)])

    return agg(h_stacked, srcoff, dst3)


def kernel(x, edge_index, W0, g0, b0, W1, g1, b1, W2, g2, b2, Wout, bout):
    n = x.shape[0]
    bn = 1000

    dst = edge_index[0].astype(jnp.int32)
    src = edge_index[1].astype(jnp.int32)

    g0 = g0.reshape(1, -1); b0 = b0.reshape(1, -1)
    g1 = g1.reshape(1, -1); b1 = b1.reshape(1, -1)
    g2 = g2.reshape(1, -1); b2 = b2.reshape(1, -1)
    bout = bout.reshape(1, -1)

    n_pad = ((n + 128 * _TILES - 1) // (128 * _TILES)) * (128 * _TILES)

    e = src.shape[0]
    ngrp = e // _TILES // _B // _NBUF
    dst3 = dst.reshape(_TILES * ngrp, _NBUF, _B)
    src3 = src.reshape(_TILES * ngrp, _NBUF, _B)
    srcoff = jnp.concatenate([src3, src3 + n], axis=0)

    h = _mm_stack(x, W0, bn)
    a = _aggregate(h.reshape(2 * n, _HALF), srcoff, dst3, n, n_pad)
    h = _ln_mm_stack(a, g0, b0, W1, bn, n)
    a = _aggregate(h.reshape(2 * n, _HALF), srcoff, dst3, n, n_pad)
    h = _ln_mm_stack(a, g1, b1, W2, bn, n)
    a = _aggregate(h.reshape(2 * n, _HALF), srcoff, dst3, n, n_pad)
    return _final(a, g2, b2, Wout, bout, bn, n)


# speedup vs baseline: 1.1963x; 1.1963x over previous
import functools

import jax
import jax.numpy as jnp
from jax import lax
from jax.experimental import pallas as pl
from jax.experimental.pallas import tpu as pltpu
from jax.experimental.pallas import tpu_sc as plsc

_EPS = 1e-5
_HALF = 128
_B = 125
_TILES = 16


def _mm_stack_body(x_ref, w_ref, out_ref):
    h = lax.dot_general(x_ref[...], w_ref[...], (((1,), (1,)), ((), ())),
                        preferred_element_type=jnp.float32)
    out_ref[0] = h[:, :_HALF]
    out_ref[1] = h[:, _HALF:]


def _mm_stack(x, w, bn):
    n, d = x.shape
    return pl.pallas_call(
        _mm_stack_body,
        grid=(n // bn,),
        in_specs=[
            pl.BlockSpec((bn, d), lambda i: (i, 0)),
            pl.BlockSpec(w.shape, lambda i: (0, 0)),
        ],
        out_specs=pl.BlockSpec((2, bn, _HALF), lambda i: (0, i, 0)),
        out_shape=jax.ShapeDtypeStruct((2, n, _HALF), jnp.float32),
    )(x, w)


def _ln_relu(a_ref, g_ref, b_ref):
    a = jnp.concatenate([a_ref[0], a_ref[1]], axis=-1)
    mu = jnp.mean(a, axis=-1, keepdims=True)
    var = jnp.mean(jnp.square(a - mu), axis=-1, keepdims=True)
    hn = (a - mu) * lax.rsqrt(var + _EPS) * g_ref[...] + b_ref[...]
    return jnp.maximum(hn, 0.0)


def _ln_mm_stack_body(a_ref, g_ref, b_ref, w_ref, out_ref):
    h = _ln_relu(a_ref, g_ref, b_ref)
    o = lax.dot_general(h, w_ref[...], (((1,), (1,)), ((), ())),
                        preferred_element_type=jnp.float32)
    out_ref[0] = o[:, :_HALF]
    out_ref[1] = o[:, _HALF:]


def _ln_mm_stack(a, g, b, w, bn, n):
    return pl.pallas_call(
        _ln_mm_stack_body,
        grid=(n // bn,),
        in_specs=[
            pl.BlockSpec((2, bn, _HALF), lambda i: (0, i, 0)),
            pl.BlockSpec(g.shape, lambda i: (0, 0)),
            pl.BlockSpec(b.shape, lambda i: (0, 0)),
            pl.BlockSpec(w.shape, lambda i: (0, 0)),
        ],
        out_specs=pl.BlockSpec((2, bn, _HALF), lambda i: (0, i, 0)),
        out_shape=jax.ShapeDtypeStruct((2, n, _HALF), jnp.float32),
    )(a, g, b, w)


def _final_body(a_ref, g_ref, b_ref, w_ref, bo_ref, out_ref):
    h = _ln_relu(a_ref, g_ref, b_ref)
    logits = lax.dot_general(h, w_ref[...], (((1,), (1,)), ((), ())),
                             preferred_element_type=jnp.float32) + bo_ref[...]
    m = jnp.max(logits, axis=-1, keepdims=True)
    lse = jnp.log(jnp.sum(jnp.exp(logits - m), axis=-1, keepdims=True)) + m
    out_ref[...] = logits - lse


def _final(a, g, b, w, bo, bn, n):
    c = w.shape[0]
    return pl.pallas_call(
        _final_body,
        grid=(n // bn,),
        in_specs=[
            pl.BlockSpec((2, bn, _HALF), lambda i: (0, i, 0)),
            pl.BlockSpec(g.shape, lambda i: (0, 0)),
            pl.BlockSpec(b.shape, lambda i: (0, 0)),
            pl.BlockSpec(w.shape, lambda i: (0, 0)),
            pl.BlockSpec(bo.shape, lambda i: (0, 0)),
        ],
        out_specs=pl.BlockSpec((bn, c), lambda i: (i, 0)),
        out_shape=jax.ShapeDtypeStruct((n, c), jnp.float32),
    )(a, g, b, w, bo)


_NBUF = 2


def _aggregate(h_stacked, srcoff, dst3, n, n_pad):
    ngrp = dst3.shape[0] // _TILES
    rows_per_tile = n_pad // _TILES
    zrows = 32
    nz = rows_per_tile // zrows
    assert rows_per_tile % zrows == 0

    mesh = plsc.VectorSubcoreMesh(core_axis_name="c", subcore_axis_name="s")

    @functools.partial(
        pl.kernel,
        mesh=mesh,
        out_type=jax.ShapeDtypeStruct((2, n_pad, _HALF), jnp.float32),
        scratch_types=[
            pltpu.VMEM((2, _NBUF, _B), jnp.int32),
            pltpu.VMEM((2, _NBUF, _B), jnp.int32),
            pltpu.VMEM((_NBUF, _B, _HALF), jnp.float32),
            pltpu.VMEM((zrows, _HALF), jnp.float32),
            pltpu.VMEM_SHARED((n_pad, _HALF), jnp.float32),
            pltpu.SemaphoreType.DMA,
        ] + [pltpu.SemaphoreType.DMA] * _NBUF,
    )
    def agg(h_hbm, src_hbm, dst_hbm, out_hbm, src_g, dst_g, rows_v, zero_v,
            acc_sh, sem_idx, *sems):
        c = lax.axis_index("c")
        s = lax.axis_index("s")
        srow0 = (c * _TILES + s) * ngrp
        drow0 = s * ngrp

        pltpu.async_copy(src_hbm.at[srow0], src_g.at[0], sem_idx)
        pltpu.async_copy(dst_hbm.at[drow0], dst_g.at[0], sem_idx)

        def zrow(i, _):
            def zcol(j, _):
                zero_v[i, pl.ds(j * 16, 16)] = jnp.zeros((16,), jnp.float32)
                return 0
            return lax.fori_loop(0, _HALF // 16, zcol, 0)
        lax.fori_loop(0, zrows, zrow, 0)
        for k in range(nz):
            pltpu.async_copy(
                zero_v,
                acc_sh.at[pl.ds(s * rows_per_tile + k * zrows, zrows)],
                sems[k % _NBUF])
        for k in range(nz):
            pltpu.make_async_copy(
                zero_v,
                acc_sh.at[pl.ds(s * rows_per_tile + k * zrows, zrows)],
                sems[k % _NBUF]).wait()

        pltpu.make_async_copy(src_hbm.at[srow0], src_g.at[0], sem_idx).wait()
        pltpu.make_async_copy(dst_hbm.at[drow0], dst_g.at[0], sem_idx).wait()
        for b in range(_NBUF):
            pltpu.async_copy(h_hbm.at[src_g.at[0, b]], rows_v.at[b], sems[b])
        pltpu.async_copy(src_hbm.at[srow0 + 1], src_g.at[1], sem_idx)
        pltpu.async_copy(dst_hbm.at[drow0 + 1], dst_g.at[1], sem_idx)
        plsc.subcore_barrier()

        def group(g, _):
            p = lax.rem(g, 2)
            q = 1 - p

            @pl.when(g + 1 < ngrp)
            def _():
                pltpu.make_async_copy(
                    src_hbm.at[srow0 + g + 1], src_g.at[q], sem_idx).wait()
                pltpu.make_async_copy(
                    dst_hbm.at[drow0 + g + 1], dst_g.at[q], sem_idx).wait()

            for b in range(_NBUF):
                pltpu.make_async_copy(
                    h_hbm.at[src_g.at[p, b]], rows_v.at[b], sems[b]).wait()
                pltpu.sync_copy(rows_v.at[b], acc_sh.at[dst_g.at[p, b]],
                                add=True)

                @pl.when(g + 1 < ngrp)
                def _():
                    pltpu.async_copy(
                        h_hbm.at[src_g.at[q, b]], rows_v.at[b], sems[b])

            @pl.when(g + 2 < ngrp)
            def _():
                pltpu.async_copy(
                    src_hbm.at[srow0 + g + 2], src_g.at[p], sem_idx)
                pltpu.async_copy(
                    dst_hbm.at[drow0 + g + 2], dst_g.at[p], sem_idx)
            return 0
        lax.fori_loop(0, ngrp, group, 0)
        plsc.subcore_barrier()

        pltpu.sync_copy(
            acc_sh.at[pl.ds(s * rows_per_tile, rows_per_tile)],
            out_hbm.at[c, pl.ds(s * rows_per_tile, rows_per_tile)])

    return agg(h_stacked, srcoff, dst3)


def kernel(x, edge_index, W0, g0, b0, W1, g1, b1, W2, g2, b2, Wout, bout):
    n = x.shape[0]
    bn = 2000

    dst = edge_index[0].astype(jnp.int32)
    src = edge_index[1].astype(jnp.int32)

    g0 = g0.reshape(1, -1); b0 = b0.reshape(1, -1)
    g1 = g1.reshape(1, -1); b1 = b1.reshape(1, -1)
    g2 = g2.reshape(1, -1); b2 = b2.reshape(1, -1)
    bout = bout.reshape(1, -1)

    n_pad = ((n + 128 * _TILES - 1) // (128 * _TILES)) * (128 * _TILES)

    e = src.shape[0]
    ngrp = e // _TILES // _B // _NBUF
    dst3 = dst.reshape(_TILES * ngrp, _NBUF, _B)
    src3 = src.reshape(_TILES * ngrp, _NBUF, _B)
    srcoff = jnp.concatenate([src3, src3 + n], axis=0)

    h = _mm_stack(x, W0, bn)
    a = _aggregate(h.reshape(2 * n, _HALF), srcoff, dst3, n, n_pad)
    h = _ln_mm_stack(a, g0, b0, W1, bn, n)
    a = _aggregate(h.reshape(2 * n, _HALF), srcoff, dst3, n, n_pad)
    h = _ln_mm_stack(a, g1, b1, W2, bn, n)
    a = _aggregate(h.reshape(2 * n, _HALF), srcoff, dst3, n, n_pad)
    return _final(a, g2, b2, Wout, bout, bn, n)
